# Initial kernel scaffold; baseline (speedup 1.0000x reference)
import jax
import jax.numpy as jnp
from jax.experimental import pallas as pl


def kernel(x, edge_index, edge_attr, W, b):
    raise NotImplementedError("write your pallas kernel here")


# speedup vs baseline: 4.3855x; 4.3855x over previous
import functools

import jax
import jax.numpy as jnp
from jax import lax
from jax.experimental import pallas as pl
from jax.experimental.pallas import tpu as pltpu
from jax.experimental.pallas import tpu_sc as plsc

NC = 2
NS = 16
L = 16
CHUNK = 128


def _rsqrt16(v):
    iv = lax.bitcast_convert_type(v, jnp.int32)
    iv = jnp.int32(0x5F3759DF) - lax.shift_right_logical(iv, 1)
    y = lax.bitcast_convert_type(iv, jnp.float32)
    for _ in range(3):
        y = y * (1.5 - 0.5 * v * y * y)
    return y


@functools.lru_cache(maxsize=None)
def _make_norm_kernel(n, nch, np_, per):
    mesh = plsc.VectorSubcoreMesh(
        core_axis_name="c", subcore_axis_name="s", num_cores=NC, num_subcores=NS
    )

    @functools.partial(
        pl.kernel,
        out_type=jax.ShapeDtypeStruct((NS, nch, CHUNK), jnp.float32),
        mesh=mesh,
        compiler_params=pltpu.CompilerParams(needs_layout_passes=False),
        scratch_types=[
            pltpu.VMEM_SHARED((np_,), jnp.float32),
            pltpu.VMEM_SHARED((np_,), jnp.float32),
            pltpu.VMEM((nch, CHUNK), jnp.int32),
            pltpu.VMEM((nch, CHUNK), jnp.int32),
            pltpu.VMEM((nch, CHUNK), jnp.float32),
            pltpu.VMEM((CHUNK,), jnp.float32),
            pltpu.VMEM((np_,), jnp.float32),
            pltpu.VMEM((per,), jnp.float32),
        ],
    )
    def norm_kernel(src_hbm, dst_hbm, w_hbm, norm_hbm,
                    deg_sh, dinv_sh, src_v, dst_v, w_v, nbuf, dinv_v, tmp_v):
        cid = lax.axis_index("c")
        sid = lax.axis_index("s")
        pltpu.sync_copy(src_hbm.at[sid], src_v)
        pltpu.sync_copy(dst_hbm.at[sid], dst_v)
        pltpu.sync_copy(w_hbm.at[sid], w_v)
        for j in range(per // L):
            tmp_v[pl.ds(j * L, L)] = jnp.zeros((L,), jnp.float32)
        pltpu.sync_copy(tmp_v, deg_sh.at[pl.ds(sid * per, per)])
        plsc.subcore_barrier()

        def deg_body(ch, _):
            pltpu.sync_copy(w_v.at[ch], deg_sh.at[dst_v.at[ch]], add=True)
            return 0

        lax.fori_loop(0, nch, deg_body, 0)
        plsc.subcore_barrier()

        pltpu.sync_copy(deg_sh.at[pl.ds(sid * per, per)], tmp_v)
        for j in range(per // L):
            v = tmp_v[pl.ds(j * L, L)]
            tmp_v[pl.ds(j * L, L)] = _rsqrt16(v)
        pltpu.sync_copy(tmp_v, dinv_sh.at[pl.ds(sid * per, per)])
        plsc.subcore_barrier()
        pltpu.sync_copy(dinv_sh, dinv_v)

        ncpc = nch // NC

        def norm_body(i, _):
            ch = cid * ncpc + i
            for j in range(CHUNK // L):
                s16 = src_v[ch, pl.ds(j * L, L)]
                d16 = dst_v[ch, pl.ds(j * L, L)]
                w16 = w_v[ch, pl.ds(j * L, L)]
                di_s = plsc.load_gather(dinv_v, [s16])
                di_d = plsc.load_gather(dinv_v, [d16])
                nbuf[pl.ds(j * L, L)] = di_s * w16 * di_d
            pltpu.sync_copy(nbuf, norm_hbm.at[sid, ch])
            return 0

        lax.fori_loop(0, ncpc, norm_body, 0)

    return norm_kernel


@functools.lru_cache(maxsize=None)
def _make_hop_kernel(n, d, nch, half, accr):
    mesh = plsc.VectorSubcoreMesh(
        core_axis_name="c", subcore_axis_name="s", num_cores=NC, num_subcores=NS
    )
    dh = d // 2
    zr = accr // NS
    dr = zr
    dr_last = half - dr * (NS - 1)

    def _blocks(total):
        out = []
        off = 0
        while off < total:
            t = min(CHUNK, total - off)
            out.append((off, t))
            off += t
        return out

    assert nch % 2 == 0 and dr_last > 0

    @functools.partial(
        pl.kernel,
        out_type=jax.ShapeDtypeStruct((2, n, dh), jnp.float32),
        mesh=mesh,
        compiler_params=pltpu.CompilerParams(needs_layout_passes=False),
        scratch_types=[
            pltpu.VMEM_SHARED((accr, dh), jnp.float32),
            pltpu.VMEM((nch, CHUNK), jnp.int32),
            pltpu.VMEM((nch, CHUNK), jnp.int32),
            pltpu.VMEM((nch, CHUNK), jnp.float32),
            pltpu.VMEM((CHUNK, dh), jnp.float32),
            pltpu.VMEM((CHUNK, dh), jnp.float32),
            pltpu.SemaphoreType.DMA,
            pltpu.SemaphoreType.DMA,
        ],
    )
    def hop_kernel(hv_hbm, src_hbm, dst_hbm, norm_hbm, out_hbm,
                   acc_sh, src_v, dloc_v, norm_v, rb0, rb1, semr0, semr1):
        cid = lax.axis_index("c")
        sid = lax.axis_index("s")
        base = cid * half
        rbs = (rb0, rb1)
        semrs = (semr0, semr1)

        pltpu.sync_copy(src_hbm.at[sid], src_v)
        pltpu.sync_copy(dst_hbm.at[sid], dloc_v)
        pltpu.sync_copy(norm_hbm.at[sid], norm_v)

        def loc_body(ch, _):
            for j in range(CHUNK // L):
                dd = dloc_v[ch, pl.ds(j * L, L)]
                owned = (dd >= base) & (dd < base + half)
                dloc_v[ch, pl.ds(j * L, L)] = jnp.where(owned, dd - base, half)
            return 0

        lax.fori_loop(0, nch, loc_body, 0)

        for hp in range(2):
            def zero_body(i, _):
                for j in range(dh // L):
                    rb0[i, pl.ds(j * L, L)] = jnp.zeros((L,), jnp.float32)
                return 0

            lax.fori_loop(0, CHUNK, zero_body, 0)
            r0 = sid * zr
            for off, t in _blocks(zr):
                pltpu.sync_copy(rb0.at[pl.ds(0, t)],
                                acc_sh.at[pl.ds(r0 + off, t)])
            plsc.subcore_barrier()

            pltpu.async_copy(hv_hbm.at[hp].at[src_v.at[0]], rb0, semr0)
            pltpu.async_copy(hv_hbm.at[hp].at[src_v.at[1]], rb1, semr1)

            def main_body(it, _, hp=hp):
                for slot in range(2):
                    ch = it * 2 + slot
                    rb = rbs[slot]
                    pltpu.make_async_copy(hv_hbm.at[hp].at[src_v.at[ch]], rb,
                                          semrs[slot]).wait()

                    def scale_body(i, _, rb=rb, ch=ch):
                        nb = plsc.load_gather(
                            norm_v,
                            [jnp.full((L,), ch, jnp.int32),
                             jnp.full((L,), i, jnp.int32)],
                        )
                        for j in range(dh // L):
                            rb[i, pl.ds(j * L, L)] = rb[i, pl.ds(j * L, L)] * nb
                        return 0

                    lax.fori_loop(0, CHUNK, scale_body, 0)
                    pltpu.sync_copy(rb, acc_sh.at[dloc_v.at[ch]], add=True)

                    @pl.when(ch + 2 < nch)
                    def _(ch=ch, slot=slot, hp=hp):
                        pltpu.async_copy(hv_hbm.at[hp].at[src_v.at[ch + 2]],
                                         rbs[slot], semrs[slot])

                return 0

            lax.fori_loop(0, nch // 2, main_body, 0)
            plsc.subcore_barrier()

            @pl.when(sid < NS - 1)
            def _(hp=hp):
                for off, t in _blocks(dr):
                    pltpu.sync_copy(
                        acc_sh.at[pl.ds(sid * dr + off, t)],
                        out_hbm.at[hp, pl.ds(base + sid * dr + off, t)])

            @pl.when(sid == NS - 1)
            def _(hp=hp):
                for off, t in _blocks(dr_last):
                    pltpu.sync_copy(
                        acc_sh.at[pl.ds(sid * dr + off, t)],
                        out_hbm.at[hp, pl.ds(base + sid * dr + off, t)])

    return hop_kernel


def _final_body(x_ref, w_ref, b_ref, o_ref):
    xv = x_ref[...]
    x = jnp.concatenate([xv[0], xv[1]], axis=-1)
    acc = jnp.dot(x, w_ref[...], preferred_element_type=jnp.float32)
    acc = acc + b_ref[...][None, :]
    acc = jnp.maximum(acc, 0.0)
    m = jnp.max(acc, axis=-1, keepdims=True)
    e = jnp.exp(acc - m)
    s = jnp.sum(e, axis=-1, keepdims=True)
    o_ref[...] = (acc - m) - jnp.log(s)


def _final(hv, W, b):
    _, n, dh = hv.shape
    d = 2 * dh
    do = W.shape[1]
    block = 2000
    return pl.pallas_call(
        _final_body,
        grid=(n // block,),
        in_specs=[
            pl.BlockSpec((2, block, dh), lambda i: (0, i, 0)),
            pl.BlockSpec((d, do), lambda i: (0, 0)),
            pl.BlockSpec((do,), lambda i: (0,)),
        ],
        out_specs=pl.BlockSpec((block, do), lambda i: (i, 0)),
        out_shape=jax.ShapeDtypeStruct((n, do), jnp.float32),
    )(hv, W, b)


def kernel(x, edge_index, edge_attr, W, b):
    n, d = x.shape
    e = edge_index.shape[1]
    e2 = e + n
    per_tile = -(-e2 // NS)
    nch = -(-per_tile // CHUNK)
    if nch % 2:
        nch += 1
    ep = NS * nch * CHUNK
    pad = ep - e2
    per = -(-n // (NS * L)) * L
    np_ = NS * per
    half = -(-n // NC)
    zr = -(-(half + 1) // NS)
    zr = -(-zr // 8) * 8
    accr = NS * zr

    idt = edge_index.dtype
    loop = jnp.arange(n, dtype=idt)
    src_e = jnp.concatenate(
        [edge_index[0], loop, jnp.zeros((pad,), idt)]).reshape(NS, nch, CHUNK)
    dst_e = jnp.concatenate(
        [edge_index[1], loop, jnp.zeros((pad,), idt)]).reshape(NS, nch, CHUNK)
    w_e = jnp.concatenate(
        [edge_attr, jnp.ones((n,), edge_attr.dtype),
         jnp.zeros((pad,), edge_attr.dtype)]).reshape(NS, nch, CHUNK)

    norm_e = _make_norm_kernel(n, nch, np_, per)(src_e, dst_e, w_e)
    xv = x.reshape(n, 2, d // 2).transpose(1, 0, 2)
    hop = _make_hop_kernel(n, d, nch, half, accr)
    h1 = hop(xv, src_e, dst_e, norm_e)
    h2 = hop(h1, src_e, dst_e, norm_e)
    return _final(h2, W, b)


# speedup vs baseline: 4.5175x; 1.0301x over previous
import functools

import jax
import jax.numpy as jnp
from jax import lax
from jax.experimental import pallas as pl
from jax.experimental.pallas import tpu as pltpu
from jax.experimental.pallas import tpu_sc as plsc

NC = 2
NS = 16
L = 16
CHUNK = 128


def _rsqrt16(v):
    iv = lax.bitcast_convert_type(v, jnp.int32)
    iv = jnp.int32(0x5F3759DF) - lax.shift_right_logical(iv, 1)
    y = lax.bitcast_convert_type(iv, jnp.float32)
    for _ in range(3):
        y = y * (1.5 - 0.5 * v * y * y)
    return y


@functools.lru_cache(maxsize=None)
def _make_norm_kernel(n, nch, np_, per, half):
    mesh = plsc.VectorSubcoreMesh(
        core_axis_name="c", subcore_axis_name="s", num_cores=NC, num_subcores=NS
    )

    @functools.partial(
        pl.kernel,
        out_type=(
            jax.ShapeDtypeStruct((NC, NS, nch, CHUNK), jnp.int32),
            jax.ShapeDtypeStruct((NC, NS, nch, CHUNK), jnp.int32),
            jax.ShapeDtypeStruct((NC, NS, nch, CHUNK), jnp.float32),
            jax.ShapeDtypeStruct((NC, NS, CHUNK), jnp.float32),
        ),
        mesh=mesh,
        compiler_params=pltpu.CompilerParams(needs_layout_passes=False),
        scratch_types=[
            pltpu.VMEM_SHARED((np_,), jnp.float32),
            pltpu.VMEM_SHARED((np_,), jnp.float32),
            pltpu.VMEM((nch, CHUNK), jnp.int32),
            pltpu.VMEM((nch, CHUNK), jnp.int32),
            pltpu.VMEM((nch, CHUNK), jnp.float32),
            pltpu.VMEM((nch, CHUNK), jnp.int32),
            pltpu.VMEM((nch, CHUNK), jnp.int32),
            pltpu.VMEM((nch, CHUNK), jnp.float32),
            pltpu.VMEM((CHUNK,), jnp.float32),
            pltpu.VMEM((np_,), jnp.float32),
            pltpu.VMEM((per,), jnp.float32),
        ],
    )
    def norm_kernel(src_hbm, dst_hbm, w_hbm,
                    psrc_hbm, pdst_hbm, pnrm_hbm, pcnt_hbm,
                    deg_sh, dinv_sh, src_v, dst_v, w_v,
                    osrc, odst, onrm, cbuf, dinv_v, tmp_v):
        cid = lax.axis_index("c")
        sid = lax.axis_index("s")
        base = cid * half
        pltpu.sync_copy(src_hbm.at[sid], src_v)
        pltpu.sync_copy(dst_hbm.at[sid], dst_v)
        pltpu.sync_copy(w_hbm.at[sid], w_v)
        for j in range(per // L):
            tmp_v[pl.ds(j * L, L)] = jnp.zeros((L,), jnp.float32)
        pltpu.sync_copy(tmp_v, deg_sh.at[pl.ds(sid * per, per)])
        plsc.subcore_barrier()

        def deg_body(ch, _):
            pltpu.sync_copy(w_v.at[ch], deg_sh.at[dst_v.at[ch]], add=True)
            return 0

        lax.fori_loop(0, nch, deg_body, 0)
        plsc.subcore_barrier()

        pltpu.sync_copy(deg_sh.at[pl.ds(sid * per, per)], tmp_v)
        for j in range(per // L):
            v = tmp_v[pl.ds(j * L, L)]
            tmp_v[pl.ds(j * L, L)] = _rsqrt16(v)
        pltpu.sync_copy(tmp_v, dinv_sh.at[pl.ds(sid * per, per)])
        plsc.subcore_barrier()
        pltpu.sync_copy(dinv_sh, dinv_v)

        def fill_body(ch, _):
            for j in range(CHUNK // L):
                osrc[ch, pl.ds(j * L, L)] = jnp.zeros((L,), jnp.int32)
                odst[ch, pl.ds(j * L, L)] = jnp.full((L,), half, jnp.int32)
                onrm[ch, pl.ds(j * L, L)] = jnp.zeros((L,), jnp.float32)
            return 0

        lax.fori_loop(0, nch, fill_body, 0)

        def comp_body(ch, ptr):
            for j in range(CHUNK // L):
                s16 = src_v[ch, pl.ds(j * L, L)]
                d16 = dst_v[ch, pl.ds(j * L, L)]
                w16 = w_v[ch, pl.ds(j * L, L)]
                di_s = plsc.load_gather(dinv_v, [s16])
                di_d = plsc.load_gather(dinv_v, [d16])
                nrm16 = di_s * w16 * di_d
                owned = (d16 >= base) & (d16 < base + half)
                incl = plsc.cumsum(jnp.where(owned, 1, 0).astype(jnp.int32))
                pos = ptr + incl - 1
                pch = pos // CHUNK
                pln = pos % CHUNK
                plsc.store_scatter(osrc, [pch, pln], s16, mask=owned)
                plsc.store_scatter(odst, [pch, pln], d16 - base, mask=owned)
                plsc.store_scatter(onrm, [pch, pln], nrm16, mask=owned)
                ptr = ptr + jnp.max(incl)
            return ptr

        cnt = lax.fori_loop(0, nch, comp_body, jnp.int32(0))
        nchs = (cnt + (CHUNK - 1)) // CHUNK
        nchs = ((nchs + 1) // 2) * 2
        nchs = jnp.maximum(nchs, 2)
        nchs_f = nchs.astype(jnp.float32)
        for j in range(CHUNK // L):
            cbuf[pl.ds(j * L, L)] = jnp.full((L,), nchs_f, jnp.float32)

        pltpu.sync_copy(osrc, psrc_hbm.at[cid, sid])
        pltpu.sync_copy(odst, pdst_hbm.at[cid, sid])
        pltpu.sync_copy(onrm, pnrm_hbm.at[cid, sid])
        pltpu.sync_copy(cbuf, pcnt_hbm.at[cid, sid])

    return norm_kernel


@functools.lru_cache(maxsize=None)
def _make_hop_kernel(n, d, nch, half, accr):
    mesh = plsc.VectorSubcoreMesh(
        core_axis_name="c", subcore_axis_name="s", num_cores=NC, num_subcores=NS
    )
    dh = d // 2
    zr = accr // NS
    dr = zr
    dr_last = half - dr * (NS - 1)

    def _blocks(total):
        out = []
        off = 0
        while off < total:
            t = min(CHUNK, total - off)
            out.append((off, t))
            off += t
        return out

    assert nch % 2 == 0 and dr_last > 0

    @functools.partial(
        pl.kernel,
        out_type=jax.ShapeDtypeStruct((2, n, dh), jnp.float32),
        mesh=mesh,
        compiler_params=pltpu.CompilerParams(needs_layout_passes=False),
        scratch_types=[
            pltpu.VMEM_SHARED((accr, dh), jnp.float32),
            pltpu.VMEM((nch, CHUNK), jnp.int32),
            pltpu.VMEM((nch, CHUNK), jnp.int32),
            pltpu.VMEM((nch, CHUNK), jnp.float32),
            pltpu.VMEM((CHUNK,), jnp.float32),
            pltpu.VMEM((CHUNK, dh), jnp.float32),
            pltpu.VMEM((CHUNK, dh), jnp.float32),
            pltpu.SemaphoreType.DMA,
            pltpu.SemaphoreType.DMA,
        ],
    )
    def hop_kernel(hv_hbm, src_hbm, dst_hbm, norm_hbm, cnt_hbm, out_hbm,
                   acc_sh, src_v, dloc_v, norm_v, cbuf, rb0, rb1, semr0, semr1):
        cid = lax.axis_index("c")
        sid = lax.axis_index("s")
        base = cid * half
        rbs = (rb0, rb1)
        semrs = (semr0, semr1)

        pltpu.sync_copy(src_hbm.at[cid, sid], src_v)
        pltpu.sync_copy(dst_hbm.at[cid, sid], dloc_v)
        pltpu.sync_copy(norm_hbm.at[cid, sid], norm_v)
        pltpu.sync_copy(cnt_hbm.at[cid, sid], cbuf)
        nchs = jnp.max(cbuf[pl.ds(0, L)]).astype(jnp.int32)
        nchs = jnp.maximum(nchs, 2)

        for hp in range(2):
            def zero_body(i, _):
                for j in range(dh // L):
                    rb0[i, pl.ds(j * L, L)] = jnp.zeros((L,), jnp.float32)
                return 0

            lax.fori_loop(0, CHUNK, zero_body, 0)
            r0 = sid * zr
            for off, t in _blocks(zr):
                pltpu.sync_copy(rb0.at[pl.ds(0, t)],
                                acc_sh.at[pl.ds(r0 + off, t)])
            plsc.subcore_barrier()

            pltpu.async_copy(hv_hbm.at[hp].at[src_v.at[0]], rb0, semr0)
            pltpu.async_copy(hv_hbm.at[hp].at[src_v.at[1]], rb1, semr1)

            def main_body(it, _, hp=hp):
                for slot in range(2):
                    ch = it * 2 + slot
                    rb = rbs[slot]
                    pltpu.make_async_copy(hv_hbm.at[hp].at[src_v.at[ch]], rb,
                                          semrs[slot]).wait()

                    def scale_body(i, _, rb=rb, ch=ch):
                        nb = plsc.load_gather(
                            norm_v,
                            [jnp.full((L,), ch, jnp.int32),
                             jnp.full((L,), i, jnp.int32)],
                        )
                        for j in range(dh // L):
                            rb[i, pl.ds(j * L, L)] = rb[i, pl.ds(j * L, L)] * nb
                        return 0

                    lax.fori_loop(0, CHUNK, scale_body, 0)
                    pltpu.sync_copy(rb, acc_sh.at[dloc_v.at[ch]], add=True)

                    @pl.when(ch + 2 < nchs)
                    def _(ch=ch, slot=slot, hp=hp):
                        pltpu.async_copy(hv_hbm.at[hp].at[src_v.at[ch + 2]],
                                         rbs[slot], semrs[slot])

                return 0

            lax.fori_loop(0, nchs // 2, main_body, 0)
            plsc.subcore_barrier()

            @pl.when(sid < NS - 1)
            def _(hp=hp):
                for off, t in _blocks(dr):
                    pltpu.sync_copy(
                        acc_sh.at[pl.ds(sid * dr + off, t)],
                        out_hbm.at[hp, pl.ds(base + sid * dr + off, t)])

            @pl.when(sid == NS - 1)
            def _(hp=hp):
                for off, t in _blocks(dr_last):
                    pltpu.sync_copy(
                        acc_sh.at[pl.ds(sid * dr + off, t)],
                        out_hbm.at[hp, pl.ds(base + sid * dr + off, t)])

    return hop_kernel


def _final_body(x_ref, w_ref, b_ref, o_ref):
    xv = x_ref[...]
    x = jnp.concatenate([xv[0], xv[1]], axis=-1)
    acc = jnp.dot(x, w_ref[...], preferred_element_type=jnp.float32)
    acc = acc + b_ref[...][None, :]
    acc = jnp.maximum(acc, 0.0)
    m = jnp.max(acc, axis=-1, keepdims=True)
    e = jnp.exp(acc - m)
    s = jnp.sum(e, axis=-1, keepdims=True)
    o_ref[...] = (acc - m) - jnp.log(s)


def _final(hv, W, b):
    _, n, dh = hv.shape
    d = 2 * dh
    do = W.shape[1]
    block = 2000
    return pl.pallas_call(
        _final_body,
        grid=(n // block,),
        in_specs=[
            pl.BlockSpec((2, block, dh), lambda i: (0, i, 0)),
            pl.BlockSpec((d, do), lambda i: (0, 0)),
            pl.BlockSpec((do,), lambda i: (0,)),
        ],
        out_specs=pl.BlockSpec((block, do), lambda i: (i, 0)),
        out_shape=jax.ShapeDtypeStruct((n, do), jnp.float32),
    )(hv, W, b)


def kernel(x, edge_index, edge_attr, W, b):
    n, d = x.shape
    e = edge_index.shape[1]
    e2 = e + n
    per_tile = -(-e2 // NS)
    nch = -(-per_tile // CHUNK)
    if nch % 2:
        nch += 1
    ep = NS * nch * CHUNK
    pad = ep - e2
    per = -(-n // (NS * L)) * L
    np_ = NS * per
    half = -(-n // NC)
    zr = -(-(half + 1) // NS)
    zr = -(-zr // 8) * 8
    accr = NS * zr

    idt = edge_index.dtype
    loop = jnp.arange(n, dtype=idt)
    src_e = jnp.concatenate(
        [edge_index[0], loop, jnp.zeros((pad,), idt)]).reshape(NS, nch, CHUNK)
    dst_e = jnp.concatenate(
        [edge_index[1], loop, jnp.zeros((pad,), idt)]).reshape(NS, nch, CHUNK)
    w_e = jnp.concatenate(
        [edge_attr, jnp.ones((n,), edge_attr.dtype),
         jnp.zeros((pad,), edge_attr.dtype)]).reshape(NS, nch, CHUNK)

    psrc, pdst, pnrm, pcnt = _make_norm_kernel(n, nch, np_, per, half)(
        src_e, dst_e, w_e)
    xv = x.reshape(n, 2, d // 2).transpose(1, 0, 2)
    hop = _make_hop_kernel(n, d, nch, half, accr)
    h1 = hop(xv, psrc, pdst, pnrm, pcnt)
    h2 = hop(h1, psrc, pdst, pnrm, pcnt)
    return _final(h2, W, b)
